# baseline (speedup 1.0000x reference)
import functools

import jax
import jax.numpy as jnp
from jax import lax
from jax.experimental import pallas as pl
from jax.experimental.pallas import tpu as pltpu
from jax.experimental.pallas import tpu_sc as plsc

N_CHECKIN = 100000
N_POI = 20000
HIDDEN = 128
BLK = 1000

C_EDGE = 2560
G = 128
NSUB = 16
BROWS = 10000
BPAD = 10240


def _pad_edges(src, dst, n_dst, e_pad):
    e = src.shape[0]
    pad = e_pad - e
    srcp = jnp.concatenate([src, jnp.zeros((pad,), jnp.int32)])
    dstp = jnp.concatenate([dst, jnp.full((pad,), n_dst, jnp.int32)])
    return srcp, dstp


def _make_agg(n_src, n_dst, e_pad):
    nblk = n_dst // (2 * BROWS)
    et = e_pad // NSUB
    nch = et // C_EDGE
    nsc = C_EDGE // G
    mesh = plsc.VectorSubcoreMesh(core_axis_name="c", subcore_axis_name="s")

    @functools.partial(
        pl.kernel,
        out_type=jax.ShapeDtypeStruct((n_dst, HIDDEN), jnp.float32),
        mesh=mesh,
        compiler_params=pltpu.CompilerParams(needs_layout_passes=False),
        scratch_types=[
            pltpu.VMEM((C_EDGE,), jnp.int32),
            pltpu.VMEM((C_EDGE,), jnp.int32),
            pltpu.VMEM((C_EDGE + 16,), jnp.int32),
            pltpu.VMEM((C_EDGE + 16,), jnp.int32),
            pltpu.VMEM((nsc, G), jnp.int32),
            pltpu.VMEM((nsc, G), jnp.int32),
            pltpu.VMEM((G, HIDDEN), jnp.float32),
            pltpu.VMEM_SHARED((BPAD, HIDDEN), jnp.float32),
            pltpu.SemaphoreType.DMA,
        ],
    )
    def agg(table, srcp, dstp, out,
            dch, sch, s1d, d1d, s2d, d2d, rows, block, sem):
        cid = lax.axis_index("c")
        sid = lax.axis_index("s")
        ebase = sid * et
        zero16f = jnp.zeros((16,), jnp.float32)
        zero16i = jnp.zeros((16,), jnp.int32)
        dum16 = jnp.full((16,), BROWS, jnp.int32)

        def zs_body(i, _):
            s1d[pl.ds(i * 16, 16)] = zero16i
            return 0
        lax.fori_loop(0, C_EDGE // 16, zs_body, 0)

        for blk in range(nblk):
            r0 = (cid * nblk + blk) * BROWS

            def zb_body(i, _):
                for k in range(HIDDEN // 16):
                    rows[i, pl.ds(k * 16, 16)] = zero16f
                return 0
            lax.fori_loop(0, G, zb_body, 0)
            for k in range(BPAD // NSUB // G):
                pltpu.sync_copy(rows, block.at[pl.ds(sid * (BPAD // NSUB)
                                                     + k * G, G)])
            plsc.subcore_barrier()

            def chunk_body(ch, _):
                base = ebase + ch * C_EDGE
                pltpu.sync_copy(dstp.at[pl.ds(base, C_EDGE)], dch)
                pltpu.sync_copy(srcp.at[pl.ds(base, C_EDGE)], sch)

                def zd_body(i, _):
                    d1d[pl.ds(i * 16, 16)] = dum16
                    return 0
                lax.fori_loop(0, C_EDGE // 16, zd_body, 0)

                def compact(i, cnt):
                    vd = dch[pl.ds(i * 16, 16)]
                    vs = sch[pl.ds(i * 16, 16)]
                    m = jnp.logical_and(vd >= r0, vd < r0 + BROWS)
                    cum = plsc.cumsum(m.astype(jnp.int32))
                    pos = jnp.where(m, cum - 1 + cnt, C_EDGE)
                    plsc.store_scatter(s1d, [pos], vs)
                    plsc.store_scatter(d1d, [pos], vd - r0)
                    return cnt + plsc.all_reduce_population_count(m)[0]
                cnt = lax.fori_loop(0, C_EDGE // 16, compact, jnp.int32(0))

                def fire(j, _):
                    @pl.when(j * G < cnt)
                    def _():
                        for k in range(G // 16):
                            s2d[j, pl.ds(k * 16, 16)] = \
                                s1d[pl.ds(j * G + k * 16, 16)]
                            d2d[j, pl.ds(k * 16, 16)] = \
                                d1d[pl.ds(j * G + k * 16, 16)]
                        pltpu.async_copy(table.at[s2d.at[j]], rows, sem).wait()
                        pltpu.sync_copy(rows, block.at[d2d.at[j]], add=True)
                    return 0
                lax.fori_loop(0, nsc, fire, 0)
                return 0
            lax.fori_loop(0, nch, chunk_body, 0)
            plsc.subcore_barrier()

            w = 80
            nchunks_wb = BROWS // w
            for k in range((nchunks_wb + NSUB - 1) // NSUB):
                idx = sid + k * NSUB

                @pl.when(idx < nchunks_wb)
                def _():
                    off = pl.multiple_of(idx * w, w)
                    pltpu.sync_copy(block.at[pl.ds(off, w)],
                                    rows.at[pl.ds(0, w)])
                    pltpu.sync_copy(rows.at[pl.ds(0, w)],
                                    out.at[pl.ds(r0 + off, w)])
            plsc.subcore_barrier()

    return agg


_EDGE_DEFS = (
    ("seq", N_CHECKIN, 614400),
    ("vtd", N_CHECKIN, 614400),
    ("vis", N_POI, 614400),
    ("sp", N_POI, 327680),
)


def _make_counts():
    mesh = plsc.VectorSubcoreMesh(core_axis_name="c", subcore_axis_name="s")
    allocs = {N_CHECKIN: 50176, N_POI: 10240}

    @functools.partial(
        pl.kernel,
        out_type=[jax.ShapeDtypeStruct((n, ), jnp.float32)
                  for _, n, _ in _EDGE_DEFS],
        mesh=mesh,
        compiler_params=pltpu.CompilerParams(needs_layout_passes=False),
        scratch_types=[
            pltpu.VMEM((C_EDGE,), jnp.int32),
            pltpu.VMEM((C_EDGE + 16,), jnp.int32),
            pltpu.VMEM((C_EDGE // G, G), jnp.int32),
            pltpu.VMEM((G,), jnp.float32),
            pltpu.VMEM((50176 // NSUB,), jnp.float32),
            pltpu.VMEM((1000,), jnp.float32),
            pltpu.VMEM_SHARED((50176,), jnp.float32),
            pltpu.VMEM_SHARED((50176,), jnp.float32),
            pltpu.VMEM_SHARED((10240,), jnp.float32),
            pltpu.VMEM_SHARED((10240,), jnp.float32),
        ],
    )
    def counts(d_seq, d_vtd, d_vis, d_sp,
               o_seq, o_vtd, o_vis, o_sp,
               dch, d1d, d2d, ones, zbuf, wbuf, c0, c1, c2, c3):
        cid = lax.axis_index("c")
        sid = lax.axis_index("s")
        one16 = jnp.ones((16,), jnp.float32)
        zero16f = jnp.zeros((16,), jnp.float32)

        def zo_body(i, _):
            ones[pl.ds(i * 16, 16)] = one16
            return 0
        lax.fori_loop(0, G // 16, zo_body, 0)

        def zz_body(i, _):
            zbuf[pl.ds(i * 16, 16)] = zero16f
            return 0
        lax.fori_loop(0, 50176 // NSUB // 16, zz_body, 0)

        for (nm, n_dst, e_pad), dst_in, out_ref, cspm in zip(
                _EDGE_DEFS, (d_seq, d_vtd, d_vis, d_sp),
                (o_seq, o_vtd, o_vis, o_sp), (c0, c1, c2, c3)):
            nhalf = n_dst // 2
            alloc = allocs[n_dst]
            share = alloc // NSUB
            et = e_pad // NSUB
            nch = et // C_EDGE
            lo = cid * nhalf
            dum16 = jnp.full((16,), nhalf, jnp.int32)

            pltpu.sync_copy(zbuf.at[pl.ds(0, share)],
                            cspm.at[pl.ds(sid * share, share)])
            plsc.subcore_barrier()

            def chunk_body(ch, _):
                base = sid * et + ch * C_EDGE
                pltpu.sync_copy(dst_in.at[pl.ds(base, C_EDGE)], dch)

                def zd_body(i, _):
                    d1d[pl.ds(i * 16, 16)] = dum16
                    return 0
                lax.fori_loop(0, C_EDGE // 16, zd_body, 0)

                def compact(i, cnt):
                    vd = dch[pl.ds(i * 16, 16)] - lo
                    m = jnp.logical_and(vd >= 0, vd < nhalf)
                    cum = plsc.cumsum(m.astype(jnp.int32))
                    pos = jnp.where(m, cum - 1 + cnt, C_EDGE)
                    plsc.store_scatter(d1d, [pos], vd)
                    return cnt + plsc.all_reduce_population_count(m)[0]
                cnt = lax.fori_loop(0, C_EDGE // 16, compact, jnp.int32(0))

                def fire(j, _):
                    @pl.when(j * G < cnt)
                    def _():
                        for k in range(G // 16):
                            d2d[j, pl.ds(k * 16, 16)] = \
                                d1d[pl.ds(j * G + k * 16, 16)]
                        pltpu.sync_copy(ones, cspm.at[d2d.at[j]], add=True)
                    return 0
                lax.fori_loop(0, C_EDGE // G, fire, 0)
                return 0
            lax.fori_loop(0, nch, chunk_body, 0)
            plsc.subcore_barrier()

            nwb = nhalf // 1000
            for k in range((nwb + NSUB - 1) // NSUB):
                idx = sid + k * NSUB

                @pl.when(idx < nwb)
                def _():
                    pltpu.sync_copy(cspm.at[pl.ds(idx * 1000, 1000)], wbuf)
                    pltpu.sync_copy(wbuf, out_ref.at[pl.ds(lo + idx * 1000,
                                                           1000)])
            plsc.subcore_barrier()

    return counts


_agg_cc = _make_agg(N_CHECKIN, N_CHECKIN, 614400)
_agg_pc = _make_agg(N_POI, N_CHECKIN, 614400)
_agg_cp = _make_agg(N_CHECKIN, N_POI, 614400)
_agg_pp = _make_agg(N_POI, N_POI, 327680)
_counts_k = _make_counts()


def _transform_body(x_ref, W_ref, b_ref, deg_ref, h_ref, hsc_ref):
    h = jnp.dot(x_ref[:], W_ref[:], preferred_element_type=jnp.float32,
                precision=lax.Precision.HIGHEST) + b_ref[:]
    deg = deg_ref[:]
    dis = jnp.where(deg > 0.0, lax.rsqrt(jnp.maximum(deg, 1e-12)), 0.0)
    h_ref[:] = h
    hsc_ref[:] = dis * h


def _transform(x, W, b, deg, n):
    row = pl.BlockSpec((BLK, HIDDEN), lambda i: (i, 0))
    return pl.pallas_call(
        _transform_body,
        grid=(n // BLK,),
        in_specs=[
            row,
            pl.BlockSpec((HIDDEN, HIDDEN), lambda i: (0, 0)),
            pl.BlockSpec((1, HIDDEN), lambda i: (0, 0)),
            pl.BlockSpec((BLK, 1), lambda i: (i, 0)),
        ],
        out_specs=[row, row],
        out_shape=[jax.ShapeDtypeStruct((n, HIDDEN), jnp.float32)] * 2,
    )(x, W, b.reshape(1, HIDDEN), deg)


def _combine_body(agg1_ref, agg2_ref, h_ref, deg_ref, cnt_ref,
                  W1_ref, W2_ref, W3_ref, b1_ref, b2_ref, pa_ref,
                  c_ref, csc_ref, *, with_prelu, with_scaled):
    deg = deg_ref[:]
    dis = jnp.where(deg > 0.0, lax.rsqrt(jnp.maximum(deg, 1e-12)), 0.0)
    invc = 1.0 / jnp.maximum(cnt_ref[:], 1.0)
    hi = lax.Precision.HIGHEST
    t = dis * jnp.dot(agg1_ref[:], W1_ref[:],
                      preferred_element_type=jnp.float32, precision=hi)
    t = t + b1_ref[:] + b2_ref[:]
    t = t + jnp.dot(invc * agg2_ref[:], W2_ref[:],
                    preferred_element_type=jnp.float32, precision=hi)
    t = t + jnp.dot(h_ref[:], W3_ref[:],
                    preferred_element_type=jnp.float32, precision=hi)
    if with_prelu:
        t = jnp.where(t >= 0.0, t, pa_ref[0, 0] * t)
    c_ref[:] = t
    if with_scaled:
        csc_ref[:] = dis * t


def _combine(agg1, agg2, h, deg, cnt, W1, W2, W3, b1, b2, pa, n,
             with_prelu, with_scaled):
    row = pl.BlockSpec((BLK, HIDDEN), lambda i: (i, 0))
    wspec = pl.BlockSpec((HIDDEN, HIDDEN), lambda i: (0, 0))
    bspec = pl.BlockSpec((1, HIDDEN), lambda i: (0, 0))
    col = pl.BlockSpec((BLK, 1), lambda i: (i, 0))
    nout = 2 if with_scaled else 1
    body = functools.partial(_combine_body, with_prelu=with_prelu,
                             with_scaled=with_scaled)
    if with_scaled:
        fn = body
    else:
        def fn(a1, a2, hh, dg, ct, w1, w2, w3, bb1, bb2, paa, c):
            body(a1, a2, hh, dg, ct, w1, w2, w3, bb1, bb2, paa, c, None)
    out = pl.pallas_call(
        fn,
        grid=(n // BLK,),
        in_specs=[row, row, row, col, col, wspec, wspec, wspec, bspec, bspec,
                  pl.BlockSpec((1, 1), lambda i: (0, 0))],
        out_specs=[row] * nout,
        out_shape=[jax.ShapeDtypeStruct((n, HIDDEN), jnp.float32)] * nout,
    )(agg1, agg2, h, deg, cnt, W1, W2, W3,
      b1.reshape(1, HIDDEN), b2.reshape(1, HIDDEN), pa.reshape(1, 1))
    return out if with_scaled else (out[0], None)


def kernel(x_checkin, x_poi, ei_seq, ei_visits, ei_visited, ei_spatial,
           Wpc, bpc, Wpp, bpp, prelu_a,
           l1_seq_W, l1_seq_b, l1_vis_Wl, l1_vis_bl, l1_vis_Wr,
           l1_vtd_Wl, l1_vtd_bl, l1_vtd_Wr, l1_sp_W, l1_sp_b,
           l2_seq_W, l2_seq_b, l2_vis_Wl, l2_vis_bl, l2_vis_Wr,
           l2_vtd_Wl, l2_vtd_bl, l2_vtd_Wr, l2_sp_W, l2_sp_b):
    pa = jnp.asarray(prelu_a, jnp.float32)
    s_seq, d_seq = _pad_edges(ei_seq[0], ei_seq[1], N_CHECKIN, 614400)
    s_vtd, d_vtd = _pad_edges(ei_visited[0], ei_visited[1], N_CHECKIN, 614400)
    s_vis, d_vis = _pad_edges(ei_visits[0], ei_visits[1], N_POI, 614400)
    s_sp, d_sp = _pad_edges(ei_spatial[0], ei_spatial[1], N_POI, 327680)

    deg_seq, cnt_vtd, cnt_vis, deg_sp = _counts_k(d_seq, d_vtd, d_vis, d_sp)
    deg_seq = deg_seq.reshape(N_CHECKIN, 1)
    cnt_vtd = cnt_vtd.reshape(N_CHECKIN, 1)
    cnt_vis = cnt_vis.reshape(N_POI, 1)
    deg_sp = deg_sp.reshape(N_POI, 1)

    hc, hc_s = _transform(x_checkin, Wpc, bpc, deg_seq, N_CHECKIN)
    hp, hp_s = _transform(x_poi, Wpp, bpp, deg_sp, N_POI)

    agg_seq = _agg_cc(hc_s, s_seq, d_seq)
    agg_vtd = _agg_pc(hp, s_vtd, d_vtd)
    agg_vis = _agg_cp(hc, s_vis, d_vis)
    agg_sp = _agg_pp(hp_s, s_sp, d_sp)

    c1, c1_s = _combine(agg_seq, agg_vtd, hc, deg_seq, cnt_vtd,
                        l1_seq_W, l1_vtd_Wl, l1_vtd_Wr, l1_seq_b, l1_vtd_bl,
                        pa, N_CHECKIN, True, True)
    p1, p1_s = _combine(agg_sp, agg_vis, hp, deg_sp, cnt_vis,
                        l1_sp_W, l1_vis_Wl, l1_vis_Wr, l1_sp_b, l1_vis_bl,
                        pa, N_POI, True, True)

    agg_seq2 = _agg_cc(c1_s, s_seq, d_seq)
    agg_vtd2 = _agg_pc(p1, s_vtd, d_vtd)
    agg_vis2 = _agg_cp(c1, s_vis, d_vis)
    agg_sp2 = _agg_pp(p1_s, s_sp, d_sp)

    c2, _ = _combine(agg_seq2, agg_vtd2, c1, deg_seq, cnt_vtd,
                     l2_seq_W, l2_vtd_Wl, l2_vtd_Wr, l2_seq_b, l2_vtd_bl,
                     pa, N_CHECKIN, False, False)
    p2, _ = _combine(agg_sp2, agg_vis2, p1, deg_sp, cnt_vis,
                     l2_sp_W, l2_vis_Wl, l2_vis_Wr, l2_sp_b, l2_vis_bl,
                     pa, N_POI, False, False)
    return (c2, p2)
